# baseline (speedup 1.0000x reference)
import jax
import jax.numpy as jnp
from jax import lax
from jax.experimental import pallas as pl
from jax.experimental.pallas import tpu as pltpu
from jax.experimental.pallas import tpu_sc as plsc

_P = 200
_RH = 3
_RW = 3
_C = 16
_SEL = 64
_H = 1024
_W = 1024

_ROWW = 128
_PIX_PER_ROW = _ROWW // _C
_NROWS = _H * _W * _C // _ROWW

_NW = 32
_PPW = 7
_PPAD = _NW * _PPW
_TAPS = _RH * _RW
_GPP = _RH * 2
_GW = 48
_IDXW = 64
_OUTW = 8
_NPAD = _NW * _OUTW


_QN = 10
_WVR = 72


def _sc_gather_body(x_hbm, grows_hbm, wvec_hbm, out_hbm,
                    gidx_v, rows_v, w_v, out_v, sem):
    nc = plsc.get_sparse_core_info().num_cores
    wid = lax.axis_index("s") * nc + lax.axis_index("c")
    pltpu.sync_copy(grows_hbm.at[pl.ds(wid * _GW, _GW)], gidx_v)
    pltpu.sync_copy(wvec_hbm.at[pl.ds(wid * _WVR, _WVR)], w_v)
    pltpu.async_copy(x_hbm.at[gidx_v], rows_v, sem).wait()
    acc = None
    for p in range(_PPW):
        acc = None
        for seg in range(_RH):
            for q in range(_QN):
                r = p * _GPP + seg * 2 + q // _PIX_PER_ROW
                c = (q % _PIX_PER_ROW) * _C
                v = rows_v[r, pl.ds(c, _C)] * w_v[p * _QN + q, :]
                acc = v if acc is None else acc + v
        out_v[p, :] = acc
    out_v[_PPW, :] = acc - acc
    pltpu.sync_copy(out_v, out_hbm.at[pl.ds(wid * _OUTW, _OUTW)])


def _score_body(sums_ref, inv_ref, s_ref):
    w = sums_ref[...]
    s = jnp.sum(w, axis=1, keepdims=True) * inv_ref[:, 0:1]
    row_i = lax.broadcasted_iota(jnp.int32, (_NPAD, 1), 0)
    slot = row_i % _OUTW
    pid = (row_i // _OUTW) * _PPW + slot
    valid = (slot < _PPW) & (pid < _P)
    s_ref[...] = jnp.broadcast_to(jnp.where(valid, s, -3e38), (_NPAD, 128))


def _thresh_body(scol_ref, srow_ref, out_ref):
    s = scol_ref[:, 0:1]
    s_row = srow_ref[0:1, :]
    ii = lax.broadcasted_iota(jnp.int32, (_NPAD, _NPAD), 0)
    jj = lax.broadcasted_iota(jnp.int32, (_NPAD, _NPAD), 1)
    beats = (s_row > s) | ((s_row == s) & (jj < ii))
    rank = jnp.sum(beats.astype(jnp.float32), axis=1, keepdims=True)

    sel = ((rank == float(_SEL - 1)) | (rank == float(_SEL))).astype(jnp.float32)
    thresh = 0.5 * jnp.sum(s * sel)
    out = jax.nn.sigmoid((s - thresh) * 100.0)
    out_ref[...] = jnp.broadcast_to(out, (_NPAD, 128))


def kernel(x, bbox, scale_ratio):
    xt = x.reshape(_NROWS, _ROWW)
    x1 = jnp.floor(bbox[:, 0] / scale_ratio[1]).astype(jnp.int32)
    y1 = jnp.floor(bbox[:, 1] / scale_ratio[0]).astype(jnp.int32)
    x2b = jnp.floor(bbox[:, 2] / scale_ratio[1]).astype(jnp.int32)
    y2 = jnp.floor(bbox[:, 3] / scale_ratio[0]).astype(jnp.int32)
    yc = jnp.clip(y1, 0, _H - _RH)
    xc = jnp.clip(x1, 0, _W - _RW)

    xb = xc // _PIX_PER_ROW
    o = xc % _PIX_PER_ROW

    dyv = jnp.arange(_RH, dtype=jnp.int32)
    jv = jnp.arange(2, dtype=jnp.int32)
    grows = ((yc[:, None, None] + dyv[None, :, None]) * (_W // _PIX_PER_ROW)
             + xb[:, None, None] + jv[None, None, :])
    grows = jnp.clip(grows, 0, _NROWS - 1)
    grows = jnp.concatenate(
        [grows.reshape(_P, _GPP),
         jnp.zeros((_PPAD - _P, _GPP), jnp.int32)], axis=0)
    grows = grows.reshape(_NW, _PPW * _GPP)
    grows = jnp.concatenate(
        [grows, jnp.zeros((_NW, _GW - _PPW * _GPP), jnp.int32)], axis=1)
    grows = grows.reshape(_NW * _GW)

    qv = jnp.arange(_QN, dtype=jnp.int32)
    wmat = ((qv[None, :] >= o[:, None])
            & (qv[None, :] <= o[:, None] + _RW - 1)).astype(jnp.float32)
    wmat = jnp.concatenate(
        [wmat, jnp.zeros((_PPAD - _P, _QN), jnp.float32)], axis=0)
    wmat = wmat.reshape(_NW, _PPW * _QN)
    wmat = jnp.concatenate(
        [wmat, jnp.zeros((_NW, _WVR - _PPW * _QN), jnp.float32)], axis=1)
    wvec = jnp.broadcast_to(
        wmat.reshape(_NW * _WVR, 1), (_NW * _WVR, _C))

    count = ((y2 - y1 + 1) * (x2b - x1 + 1) * _C).astype(jnp.float32)
    inv_rows = jnp.concatenate(
        [jnp.concatenate([1.0 / count,
                          jnp.ones((_PPAD - _P,), jnp.float32)]).reshape(_NW, _PPW),
         jnp.ones((_NW, _OUTW - _PPW), jnp.float32)], axis=1).reshape(_NPAD, 1)
    inv_b = jnp.broadcast_to(inv_rows, (_NPAD, 128))

    mesh = plsc.VectorSubcoreMesh(core_axis_name="c", subcore_axis_name="s")
    sums = pl.kernel(
        _sc_gather_body,
        out_type=jax.ShapeDtypeStruct((_NPAD, _C), jnp.float32),
        mesh=mesh,
        scratch_types=[
            pltpu.VMEM((_GW,), jnp.int32),
            pltpu.VMEM((_GW, _ROWW), jnp.float32),
            pltpu.VMEM((_WVR, _C), jnp.float32),
            pltpu.VMEM((_OUTW, _C), jnp.float32),
            pltpu.SemaphoreType.DMA,
        ],
    )(xt, grows, wvec)

    scol = pl.pallas_call(
        _score_body,
        in_specs=[
            pl.BlockSpec((_NPAD, _C), lambda: (0, 0)),
            pl.BlockSpec((_NPAD, 128), lambda: (0, 0)),
        ],
        out_specs=pl.BlockSpec((_NPAD, 128), lambda: (0, 0)),
        out_shape=jax.ShapeDtypeStruct((_NPAD, 128), jnp.float32),
    )(sums, inv_b)

    srow = jnp.broadcast_to(scol[:, 0].reshape(1, _NPAD), (8, _NPAD))

    out = pl.pallas_call(
        _thresh_body,
        in_specs=[
            pl.BlockSpec((_NPAD, 128), lambda: (0, 0)),
            pl.BlockSpec((8, _NPAD), lambda: (0, 0)),
        ],
        out_specs=pl.BlockSpec((_NPAD, 128), lambda: (0, 0)),
        out_shape=jax.ShapeDtypeStruct((_NPAD, 128), jnp.float32),
    )(scol, srow)

    rows = (jnp.arange(_P, dtype=jnp.int32) // _PPW) * _OUTW + (
        jnp.arange(_P, dtype=jnp.int32) % _PPW)
    return out[rows, 0].reshape(_P, 1, 1, 1, 1)


# speedup vs baseline: 4.7533x; 4.7533x over previous
import jax
import jax.numpy as jnp
from jax.experimental import pallas as pl
from jax.experimental.pallas import tpu as pltpu

_P = 200
_RH = 3
_RW = 3
_C = 16
_SEL = 64


def _selector_kernel(idx_ref, x_hbm, inv_ref, out_ref, patches, sem):
    copies = []
    for i in range(_P):
        y = idx_ref[0, i]
        c = idx_ref[1, i]
        cp = pltpu.make_async_copy(
            x_hbm.at[pl.ds(y, _RH), pl.ds(c, _RW), :],
            patches.at[i],
            sem,
        )
        cp.start()
        copies.append(cp)
    for cp in copies:
        cp.wait()

    w = patches[...]
    sums = jnp.sum(w, axis=(1, 2, 3), keepdims=True)
    s_col = sums.reshape(_P, 1) * inv_ref[:, 0:1]
    out_ref[...] = jnp.broadcast_to(s_col, (_P, 128))


def _thresh_kernel(scol_ref, srow_ref, out_ref):
    s_col = scol_ref[:, 0:1]
    s_row = srow_ref[0:1, :]
    ii = jax.lax.broadcasted_iota(jnp.int32, (_P, _P), 0)
    jj = jax.lax.broadcasted_iota(jnp.int32, (_P, _P), 1)
    beats = (s_row > s_col) | ((s_row == s_col) & (jj < ii))
    rank = jnp.sum(beats.astype(jnp.float32), axis=1, keepdims=True)

    sel = ((rank == float(_SEL - 1)) | (rank == float(_SEL))).astype(jnp.float32)
    thresh = 0.5 * jnp.sum(s_col * sel)
    out = jax.nn.sigmoid((s_col - thresh) * 100.0)
    out_ref[...] = jnp.broadcast_to(out, (_P, 128))


def kernel(x, bbox, scale_ratio):
    x3 = x.reshape(1024, 1024, _C)
    x1 = jnp.floor(bbox[:, 0] / scale_ratio[1]).astype(jnp.int32)
    y1 = jnp.floor(bbox[:, 1] / scale_ratio[0]).astype(jnp.int32)
    x2 = jnp.floor(bbox[:, 2] / scale_ratio[1]).astype(jnp.int32)
    y2 = jnp.floor(bbox[:, 3] / scale_ratio[0]).astype(jnp.int32)
    yc = jnp.clip(y1, 0, x3.shape[0] - _RH)
    xc = jnp.clip(x1, 0, x3.shape[1] - _RW)
    idx = jnp.stack([yc, xc]).astype(jnp.int32)
    count = ((y2 - y1 + 1) * (x2 - x1 + 1) * _C).astype(jnp.float32)
    inv = jnp.broadcast_to((1.0 / count)[:, None], (_P, 128))

    scol = pl.pallas_call(
        _selector_kernel,
        grid_spec=pltpu.PrefetchScalarGridSpec(
            num_scalar_prefetch=1,
            grid=(1,),
            in_specs=[
                pl.BlockSpec(memory_space=pl.ANY),
                pl.BlockSpec((_P, 128), lambda i, idx_ref: (0, 0)),
            ],
            out_specs=pl.BlockSpec((_P, 128), lambda i, idx_ref: (0, 0)),
            scratch_shapes=[
                pltpu.VMEM((_P, _RH, _RW, _C), jnp.float32),
                pltpu.SemaphoreType.DMA,
            ],
        ),
        out_shape=jax.ShapeDtypeStruct((_P, 128), jnp.float32),
    )(idx, x3, inv)

    srow = jnp.broadcast_to(scol[:, 0].reshape(1, _P), (8, _P))

    out = pl.pallas_call(
        _thresh_kernel,
        in_specs=[
            pl.BlockSpec((_P, 128), lambda: (0, 0)),
            pl.BlockSpec((8, _P), lambda: (0, 0)),
        ],
        out_specs=pl.BlockSpec((_P, 128), lambda: (0, 0)),
        out_shape=jax.ShapeDtypeStruct((_P, 128), jnp.float32),
    )(scol, srow)
    return out[:, 0].reshape(_P, 1, 1, 1, 1)
